# Initial kernel scaffold; baseline (speedup 1.0000x reference)
import jax
import jax.numpy as jnp
from jax.experimental import pallas as pl


def kernel(x, edge_index, W, b, ln_gamma, ln_beta):
    raise NotImplementedError("write your pallas kernel here")


# speedup vs baseline: 10.7355x; 10.7355x over previous
import functools

import jax
import jax.numpy as jnp
from jax import lax
from jax.experimental import pallas as pl
from jax.experimental.pallas import tpu as pltpu
from jax.experimental.pallas import tpu_sc as plsc

N = 10000
E = 320000
D = 128

NCORES = 2
NSUB = 16
NTILES = NCORES * NSUB
CHUNK = 128
CPT = 80
NCHUNKS = NTILES * CPT
EPAD = NCHUNKS * CHUNK
NPAD = 10112
DEGP = 10240

_mesh = plsc.VectorSubcoreMesh(core_axis_name="c", subcore_axis_name="s")


@functools.partial(
    pl.kernel,
    out_type=jax.ShapeDtypeStruct((NCORES, DEGP), jnp.float32),
    mesh=_mesh,
    scratch_types=[
        pltpu.VMEM_SHARED((DEGP,), jnp.float32),
        pltpu.VMEM((CPT, CHUNK), jnp.int32),
        pltpu.VMEM((CHUNK,), jnp.float32),
        pltpu.VMEM((640,), jnp.float32),
    ],
)
def _deg_kernel(dst_hbm, out_hbm, deg_sh, idx_v, ones_v, z_v):
    c = lax.axis_index("c")
    s = lax.axis_index("s")
    wid = c * NSUB + s

    def fill(i, _):
        z_v[pl.ds(i * 16, 16)] = jnp.zeros((16,), jnp.float32)
        return 0
    lax.fori_loop(0, 640 // 16, fill, 0)

    def fill1(i, _):
        ones_v[pl.ds(i * 16, 16)] = jnp.ones((16,), jnp.float32)
        return 0
    lax.fori_loop(0, CHUNK // 16, fill1, 0)

    pltpu.sync_copy(z_v, deg_sh.at[pl.ds(s * 640, 640)])
    pltpu.sync_copy(dst_hbm.at[pl.ds(wid * CPT, CPT)], idx_v)
    plsc.subcore_barrier()

    def body(j, _):
        pltpu.sync_copy(ones_v, deg_sh.at[idx_v.at[j]], add=True)
        return 0
    lax.fori_loop(0, CPT, body, 0)

    plsc.subcore_barrier()
    pltpu.sync_copy(deg_sh.at[pl.ds(s * 640, 640)], out_hbm.at[c, pl.ds(s * 640, 640)])


@functools.partial(
    pl.kernel,
    out_type=jax.ShapeDtypeStruct((NCORES, NPAD, D), jnp.float32),
    mesh=_mesh,
    scratch_types=[
        pltpu.VMEM_SHARED((NPAD, D), jnp.float32),
        pltpu.VMEM((CPT, CHUNK), jnp.int32),
        pltpu.VMEM((CPT, CHUNK), jnp.int32),
        pltpu.VMEM((CHUNK, D), jnp.float32),
        pltpu.VMEM((8, D), jnp.float32),
        pltpu.SemaphoreType.DMA,
    ],
)
def _acc_kernel(h3_hbm, src_hbm, dst_hbm, out_hbm, acc_sh, sidx_v, didx_v,
                rows_v, zb_v, sem):
    c = lax.axis_index("c")
    s = lax.axis_index("s")
    wid = c * NSUB + s

    def zfill(i, _):
        def zrow(k, _):
            zb_v[i, pl.ds(k * 16, 16)] = jnp.zeros((16,), jnp.float32)
            return 0
        lax.fori_loop(0, D // 16, zrow, 0)
        return 0
    lax.fori_loop(0, 8, zfill, 0)

    def zacc(i, _):
        pltpu.sync_copy(zb_v, acc_sh.at[pl.ds(s * (NPAD // NSUB) + i * 8, 8)])
        return 0
    lax.fori_loop(0, NPAD // NSUB // 8, zacc, 0)

    pltpu.sync_copy(src_hbm.at[pl.ds(wid * CPT, CPT)], sidx_v)
    pltpu.sync_copy(dst_hbm.at[pl.ds(wid * CPT, CPT)], didx_v)
    plsc.subcore_barrier()

    def body(j, _):
        pltpu.async_copy(h3_hbm.at[sidx_v.at[j]], rows_v, sem).wait()
        pltpu.sync_copy(rows_v, acc_sh.at[didx_v.at[j]], add=True)
        return 0
    lax.fori_loop(0, CPT, body, 0)

    plsc.subcore_barrier()
    rpt = NPAD // NSUB
    pltpu.sync_copy(acc_sh.at[pl.ds(s * rpt, rpt)], out_hbm.at[c, pl.ds(s * rpt, rpt)])


def _h3_body(x_ref, w_ref, degp_ref, g_ref, bt_ref, o_ref):
    xb = x_ref[...]
    mean = jnp.mean(xb, axis=1, keepdims=True)
    xc = xb - mean
    var = jnp.mean(xc * xc, axis=1, keepdims=True)
    h = xc * lax.rsqrt(var + 1e-5) * g_ref[...] + bt_ref[...]
    h2 = jnp.dot(h, w_ref[...], preferred_element_type=jnp.float32)
    deg = jnp.sum(degp_ref[...], axis=0) + 1.0
    o_ref[...] = h2 * lax.rsqrt(deg)[:, None]


def _out_body(x_ref, acc_ref, h3_ref, degp_ref, b_ref, o_ref):
    acc = acc_ref[0] + acc_ref[1] + h3_ref[...]
    deg = jnp.sum(degp_ref[...], axis=0) + 1.0
    g = lax.rsqrt(deg)[:, None] * acc + b_ref[...]
    o_ref[...] = jnp.maximum(x_ref[...] + g, 0.0)


_RB = 2048
_NB = 5


def kernel(x, edge_index, W, b, ln_gamma, ln_beta):
    ei = edge_index.astype(jnp.int32)
    src = jnp.concatenate([ei[0], jnp.zeros((EPAD - E,), jnp.int32)])
    dst = jnp.concatenate([ei[1], jnp.full((EPAD - E,), N, jnp.int32)])
    src2d = src.reshape(NCHUNKS, CHUNK)
    dst2d = dst.reshape(NCHUNKS, CHUNK)

    deg_parts = _deg_kernel(dst2d)

    h3 = pl.pallas_call(
        _h3_body,
        grid=(_NB,),
        in_specs=[
            pl.BlockSpec((_RB, D), lambda i: (i, 0)),
            pl.BlockSpec((D, D), lambda i: (0, 0)),
            pl.BlockSpec((NCORES, _RB), lambda i: (0, i)),
            pl.BlockSpec((1, D), lambda i: (0, 0)),
            pl.BlockSpec((1, D), lambda i: (0, 0)),
        ],
        out_specs=pl.BlockSpec((_RB, D), lambda i: (i, 0)),
        out_shape=jax.ShapeDtypeStruct((N, D), jnp.float32),
    )(x, W, deg_parts, ln_gamma.reshape(1, D), ln_beta.reshape(1, D))

    acc_parts = _acc_kernel(h3, src2d, dst2d)

    out = pl.pallas_call(
        _out_body,
        grid=(_NB,),
        in_specs=[
            pl.BlockSpec((_RB, D), lambda i: (i, 0)),
            pl.BlockSpec((NCORES, _RB, D), lambda i: (0, i, 0)),
            pl.BlockSpec((_RB, D), lambda i: (i, 0)),
            pl.BlockSpec((NCORES, _RB), lambda i: (0, i)),
            pl.BlockSpec((1, D), lambda i: (0, 0)),
        ],
        out_specs=pl.BlockSpec((_RB, D), lambda i: (i, 0)),
        out_shape=jax.ShapeDtypeStruct((N, D), jnp.float32),
    )(x, acc_parts, h3, deg_parts, b.reshape(1, D))

    return out


# speedup vs baseline: 28.9812x; 2.6996x over previous
import functools

import jax
import jax.numpy as jnp
from jax import lax
from jax.experimental import pallas as pl
from jax.experimental.pallas import tpu as pltpu
from jax.experimental.pallas import tpu_sc as plsc

N = 10000
E = 320000
D = 128

NCORES = 2
NSUB = 16
NTILES = NCORES * NSUB
CHUNK = 128
CPT = 80
NCHUNKS = NTILES * CPT
EPAD = NCHUNKS * CHUNK
NPAD = 10112
DEGP = 10240

_mesh = plsc.VectorSubcoreMesh(core_axis_name="c", subcore_axis_name="s")


@functools.partial(
    pl.kernel,
    out_type=jax.ShapeDtypeStruct((NCORES, DEGP), jnp.float32),
    mesh=_mesh,
    scratch_types=[
        pltpu.VMEM_SHARED((DEGP,), jnp.float32),
        pltpu.VMEM((CPT, CHUNK), jnp.int32),
        pltpu.VMEM((CHUNK,), jnp.float32),
        pltpu.VMEM((640,), jnp.float32),
    ],
)
def _deg_kernel(dst_hbm, out_hbm, deg_sh, idx_v, ones_v, z_v):
    c = lax.axis_index("c")
    s = lax.axis_index("s")
    wid = c * NSUB + s

    def fill(i, _):
        z_v[pl.ds(i * 16, 16)] = jnp.zeros((16,), jnp.float32)
        return 0
    lax.fori_loop(0, 640 // 16, fill, 0)

    def fill1(i, _):
        ones_v[pl.ds(i * 16, 16)] = jnp.ones((16,), jnp.float32)
        return 0
    lax.fori_loop(0, CHUNK // 16, fill1, 0)

    pltpu.sync_copy(z_v, deg_sh.at[pl.ds(s * 640, 640)])
    pltpu.sync_copy(dst_hbm.at[pl.ds(wid * CPT, CPT)], idx_v)
    plsc.subcore_barrier()

    def body(j, _):
        pltpu.sync_copy(ones_v, deg_sh.at[idx_v.at[j]], add=True)
        return 0
    lax.fori_loop(0, CPT, body, 0)

    plsc.subcore_barrier()
    pltpu.sync_copy(deg_sh.at[pl.ds(s * 640, 640)], out_hbm.at[c, pl.ds(s * 640, 640)])


@functools.partial(
    pl.kernel,
    out_type=jax.ShapeDtypeStruct((NCORES, NPAD, D), jnp.float32),
    mesh=_mesh,
    scratch_types=[
        pltpu.VMEM_SHARED((NPAD, D), jnp.float32),
        pltpu.VMEM((CPT, CHUNK), jnp.int32),
        pltpu.VMEM((CPT, CHUNK), jnp.int32),
        pltpu.VMEM((CHUNK, D), jnp.float32),
        pltpu.VMEM((8, D), jnp.float32),
        pltpu.SemaphoreType.DMA,
    ],
)
def _acc_kernel(h3_hbm, src_hbm, dst_hbm, out_hbm, acc_sh, sidx_v, didx_v,
                rows_v, zb_v, sem):
    c = lax.axis_index("c")
    s = lax.axis_index("s")
    wid = c * NSUB + s

    def zfill(i, _):
        def zrow(k, _):
            zb_v[i, pl.ds(k * 16, 16)] = jnp.zeros((16,), jnp.float32)
            return 0
        lax.fori_loop(0, D // 16, zrow, 0)
        return 0
    lax.fori_loop(0, 8, zfill, 0)

    def zacc(i, _):
        pltpu.sync_copy(zb_v, acc_sh.at[pl.ds(s * (NPAD // NSUB) + i * 8, 8)])
        return 0
    lax.fori_loop(0, NPAD // NSUB // 8, zacc, 0)

    pltpu.sync_copy(src_hbm.at[pl.ds(wid * CPT, CPT)], sidx_v)
    pltpu.sync_copy(dst_hbm.at[pl.ds(wid * CPT, CPT)], didx_v)
    plsc.subcore_barrier()

    def body(j, _):
        pltpu.async_copy(h3_hbm.at[sidx_v.at[j]], rows_v, sem).wait()
        pltpu.sync_copy(rows_v, acc_sh.at[didx_v.at[j]], add=True)
        return 0
    lax.fori_loop(0, CPT, body, 0)

    plsc.subcore_barrier()
    rpt = NPAD // NSUB
    pltpu.sync_copy(acc_sh.at[pl.ds(s * rpt, rpt)], out_hbm.at[c, pl.ds(s * rpt, rpt)])


def _h3_body(x_ref, w_ref, degp_ref, g_ref, bt_ref, o_ref):
    xb = x_ref[...]
    mean = jnp.mean(xb, axis=1, keepdims=True)
    xc = xb - mean
    var = jnp.mean(xc * xc, axis=1, keepdims=True)
    h = xc * lax.rsqrt(var + 1e-5) * g_ref[...] + bt_ref[...]
    h2 = jnp.dot(h, w_ref[...], preferred_element_type=jnp.float32)
    deg = jnp.sum(degp_ref[...], axis=0) + 1.0
    o_ref[...] = h2 * lax.rsqrt(deg)[:, None]


def _out_body(x_ref, acc_ref, h3_ref, degp_ref, b_ref, o_ref):
    acc = acc_ref[0] + acc_ref[1] + h3_ref[...]
    deg = jnp.sum(degp_ref[...], axis=0) + 1.0
    g = lax.rsqrt(deg)[:, None] * acc + b_ref[...]
    o_ref[...] = jnp.maximum(x_ref[...] + g, 0.0)


_RB = 2048
_NB = 5


def kernel(x, edge_index, W, b, ln_gamma, ln_beta):
    ei = edge_index.astype(jnp.int32)
    pad_i = jnp.arange(EPAD - E, dtype=jnp.int32)
    src = jnp.concatenate([ei[0], pad_i % N])
    dst = jnp.concatenate([ei[1], N + pad_i % (NPAD - N)])
    src2d = src.reshape(NCHUNKS, CHUNK)
    dst2d = dst.reshape(NCHUNKS, CHUNK)

    deg_parts = _deg_kernel(dst2d)

    h3 = pl.pallas_call(
        _h3_body,
        grid=(_NB,),
        in_specs=[
            pl.BlockSpec((_RB, D), lambda i: (i, 0)),
            pl.BlockSpec((D, D), lambda i: (0, 0)),
            pl.BlockSpec((NCORES, _RB), lambda i: (0, i)),
            pl.BlockSpec((1, D), lambda i: (0, 0)),
            pl.BlockSpec((1, D), lambda i: (0, 0)),
        ],
        out_specs=pl.BlockSpec((_RB, D), lambda i: (i, 0)),
        out_shape=jax.ShapeDtypeStruct((N, D), jnp.float32),
    )(x, W, deg_parts, ln_gamma.reshape(1, D), ln_beta.reshape(1, D))

    acc_parts = _acc_kernel(h3, src2d, dst2d)

    out = pl.pallas_call(
        _out_body,
        grid=(_NB,),
        in_specs=[
            pl.BlockSpec((_RB, D), lambda i: (i, 0)),
            pl.BlockSpec((NCORES, _RB, D), lambda i: (0, i, 0)),
            pl.BlockSpec((_RB, D), lambda i: (i, 0)),
            pl.BlockSpec((NCORES, _RB), lambda i: (0, i)),
            pl.BlockSpec((1, D), lambda i: (0, 0)),
        ],
        out_specs=pl.BlockSpec((_RB, D), lambda i: (i, 0)),
        out_shape=jax.ShapeDtypeStruct((N, D), jnp.float32),
    )(x, acc_parts, h3, deg_parts, b.reshape(1, D))

    return out


# speedup vs baseline: 39.4974x; 1.3629x over previous
import functools

import jax
import jax.numpy as jnp
from jax import lax
from jax.experimental import pallas as pl
from jax.experimental.pallas import tpu as pltpu
from jax.experimental.pallas import tpu_sc as plsc

N = 10000
E = 320000
D = 128

NCORES = 2
NSUB = 16
NTILES = NCORES * NSUB
CHUNK = 128
CPT = 80
NCHUNKS = NTILES * CPT
EPAD = NCHUNKS * CHUNK
NPAD = 10112
DEGP = 10240

_mesh = plsc.VectorSubcoreMesh(core_axis_name="c", subcore_axis_name="s")


@functools.partial(
    pl.kernel,
    out_type=jax.ShapeDtypeStruct((NCORES, DEGP), jnp.float32),
    mesh=_mesh,
    scratch_types=[
        pltpu.VMEM_SHARED((DEGP,), jnp.float32),
        pltpu.VMEM((CPT, CHUNK), jnp.int32),
        pltpu.VMEM((CHUNK,), jnp.float32),
        pltpu.VMEM((640,), jnp.float32),
    ],
)
def _deg_kernel(dst_hbm, out_hbm, deg_sh, idx_v, ones_v, z_v):
    c = lax.axis_index("c")
    s = lax.axis_index("s")
    wid = c * NSUB + s

    def fill(i, _):
        z_v[pl.ds(i * 16, 16)] = jnp.zeros((16,), jnp.float32)
        return 0
    lax.fori_loop(0, 640 // 16, fill, 0)

    def fill1(i, _):
        ones_v[pl.ds(i * 16, 16)] = jnp.ones((16,), jnp.float32)
        return 0
    lax.fori_loop(0, CHUNK // 16, fill1, 0)

    pltpu.sync_copy(z_v, deg_sh.at[pl.ds(s * 640, 640)])
    pltpu.sync_copy(dst_hbm.at[pl.ds(wid * CPT, CPT)], idx_v)
    plsc.subcore_barrier()

    def body(j, _):
        pltpu.sync_copy(ones_v, deg_sh.at[idx_v.at[j]], add=True)
        return 0
    lax.fori_loop(0, CPT, body, 0)

    plsc.subcore_barrier()
    pltpu.sync_copy(deg_sh.at[pl.ds(s * 640, 640)], out_hbm.at[c, pl.ds(s * 640, 640)])


@functools.partial(
    pl.kernel,
    out_type=jax.ShapeDtypeStruct((NCORES, NPAD, D), jnp.float32),
    mesh=_mesh,
    scratch_types=[
        pltpu.VMEM_SHARED((NPAD, D), jnp.float32),
        pltpu.VMEM((CPT // 2, CHUNK), jnp.int32),
        pltpu.VMEM((CPT // 2, CHUNK), jnp.int32),
        pltpu.VMEM((2, CHUNK, D), jnp.float32),
        pltpu.VMEM((8, D), jnp.float32),
        pltpu.SemaphoreType.DMA,
        pltpu.SemaphoreType.DMA,
    ],
)
def _acc_kernel(h3_hbm, src_hbm, dst_hbm, out_hbm, acc_sh, sidx_v, didx_v,
                rows_v, zb_v, sem0, sem1):
    c = lax.axis_index("c")
    s = lax.axis_index("s")
    wid = c * NSUB + s

    def zfill(i, _):
        def zrow(k, _):
            zb_v[i, pl.ds(k * 16, 16)] = jnp.zeros((16,), jnp.float32)
            return 0
        lax.fori_loop(0, D // 16, zrow, 0)
        return 0
    lax.fori_loop(0, 8, zfill, 0)

    def zacc(i, _):
        pltpu.sync_copy(zb_v, acc_sh.at[pl.ds(s * (NPAD // NSUB) + i * 8, 8)])
        return 0
    lax.fori_loop(0, NPAD // NSUB // 8, zacc, 0)

    plsc.subcore_barrier()

    sems = (sem0, sem1)
    half = CPT // 2
    for h in range(2):
        pltpu.sync_copy(src_hbm.at[pl.ds(wid * CPT + h * half, half)], sidx_v)
        pltpu.sync_copy(dst_hbm.at[pl.ds(wid * CPT + h * half, half)], didx_v)
        for b in range(2):
            pltpu.async_copy(h3_hbm.at[sidx_v.at[b]], rows_v.at[b], sems[b])

        def body(i, _):
            for b in range(2):
                j = 2 * i + b
                pltpu.make_async_copy(
                    h3_hbm.at[pl.ds(0, CHUNK)], rows_v.at[b], sems[b]).wait()
                pltpu.sync_copy(rows_v.at[b], acc_sh.at[didx_v.at[j]], add=True)
                jn = lax.rem(j + 2, half)
                pltpu.async_copy(h3_hbm.at[sidx_v.at[jn]], rows_v.at[b], sems[b])
            return 0
        lax.fori_loop(0, half // 2, body, 0)
        for b in range(2):
            pltpu.make_async_copy(
                h3_hbm.at[pl.ds(0, CHUNK)], rows_v.at[b], sems[b]).wait()

    plsc.subcore_barrier()
    rpt = NPAD // NSUB
    pltpu.sync_copy(acc_sh.at[pl.ds(s * rpt, rpt)], out_hbm.at[c, pl.ds(s * rpt, rpt)])


def _h3_body(x_ref, w_ref, degp_ref, g_ref, bt_ref, o_ref):
    xb = x_ref[...]
    mean = jnp.mean(xb, axis=1, keepdims=True)
    xc = xb - mean
    var = jnp.mean(xc * xc, axis=1, keepdims=True)
    h = xc * lax.rsqrt(var + 1e-5) * g_ref[...] + bt_ref[...]
    h2 = jnp.dot(h, w_ref[...], preferred_element_type=jnp.float32)
    deg = jnp.sum(degp_ref[...], axis=0) + 1.0
    o_ref[...] = h2 * lax.rsqrt(deg)[:, None]


def _out_body(x_ref, acc_ref, h3_ref, degp_ref, b_ref, o_ref):
    acc = acc_ref[0] + acc_ref[1] + h3_ref[...]
    deg = jnp.sum(degp_ref[...], axis=0) + 1.0
    g = lax.rsqrt(deg)[:, None] * acc + b_ref[...]
    o_ref[...] = jnp.maximum(x_ref[...] + g, 0.0)


_RB = 2048
_NB = 5


def kernel(x, edge_index, W, b, ln_gamma, ln_beta):
    ei = edge_index.astype(jnp.int32)
    pad_i = jnp.arange(EPAD - E, dtype=jnp.int32)
    src = jnp.concatenate([ei[0], pad_i % N])
    dst = jnp.concatenate([ei[1], N + pad_i % (NPAD - N)])
    src2d = src.reshape(NCHUNKS, CHUNK)
    dst2d = dst.reshape(NCHUNKS, CHUNK)

    deg_parts = _deg_kernel(dst2d)

    h3 = pl.pallas_call(
        _h3_body,
        grid=(_NB,),
        in_specs=[
            pl.BlockSpec((_RB, D), lambda i: (i, 0)),
            pl.BlockSpec((D, D), lambda i: (0, 0)),
            pl.BlockSpec((NCORES, _RB), lambda i: (0, i)),
            pl.BlockSpec((1, D), lambda i: (0, 0)),
            pl.BlockSpec((1, D), lambda i: (0, 0)),
        ],
        out_specs=pl.BlockSpec((_RB, D), lambda i: (i, 0)),
        out_shape=jax.ShapeDtypeStruct((N, D), jnp.float32),
    )(x, W, deg_parts, ln_gamma.reshape(1, D), ln_beta.reshape(1, D))

    acc_parts = _acc_kernel(h3, src2d, dst2d)

    out = pl.pallas_call(
        _out_body,
        grid=(_NB,),
        in_specs=[
            pl.BlockSpec((_RB, D), lambda i: (i, 0)),
            pl.BlockSpec((NCORES, _RB, D), lambda i: (0, i, 0)),
            pl.BlockSpec((_RB, D), lambda i: (i, 0)),
            pl.BlockSpec((NCORES, _RB), lambda i: (0, i)),
            pl.BlockSpec((1, D), lambda i: (0, 0)),
        ],
        out_specs=pl.BlockSpec((_RB, D), lambda i: (i, 0)),
        out_shape=jax.ShapeDtypeStruct((N, D), jnp.float32),
    )(x, acc_parts, h3, deg_parts, b.reshape(1, D))

    return out


# speedup vs baseline: 39.8778x; 1.0096x over previous
import functools

import jax
import jax.numpy as jnp
import numpy as np
from jax import lax
from jax.experimental import pallas as pl
from jax.experimental.pallas import tpu as pltpu
from jax.experimental.pallas import tpu_sc as plsc

N = 10000
E = 320000
D = 128

NCORES = 2
NSUB = 16
NTILES = NCORES * NSUB
CHUNK = 128
CPT = 80
NCHUNKS = NTILES * CPT
EPAD = NCHUNKS * CHUNK
NPAD = 10112
DEGP = 10240

_mesh = plsc.VectorSubcoreMesh(core_axis_name="c", subcore_axis_name="s")


@functools.partial(
    pl.kernel,
    out_type=jax.ShapeDtypeStruct((NCORES, DEGP), jnp.float32),
    mesh=_mesh,
    scratch_types=[
        pltpu.VMEM_SHARED((DEGP,), jnp.float32),
        pltpu.VMEM((CPT, CHUNK), jnp.int32),
        pltpu.VMEM((CHUNK,), jnp.float32),
        pltpu.VMEM((640,), jnp.float32),
    ],
)
def _deg_kernel(dst_hbm, out_hbm, deg_sh, idx_v, ones_v, z_v):
    c = lax.axis_index("c")
    s = lax.axis_index("s")
    wid = c * NSUB + s

    def fill(i, _):
        z_v[pl.ds(i * 16, 16)] = jnp.zeros((16,), jnp.float32)
        return 0
    lax.fori_loop(0, 640 // 16, fill, 0)

    def fill1(i, _):
        ones_v[pl.ds(i * 16, 16)] = jnp.ones((16,), jnp.float32)
        return 0
    lax.fori_loop(0, CHUNK // 16, fill1, 0)

    pltpu.sync_copy(z_v, deg_sh.at[pl.ds(s * 640, 640)])
    pltpu.sync_copy(dst_hbm.at[pl.ds(wid * CPT, CPT)], idx_v)
    plsc.subcore_barrier()

    def body(j, _):
        pltpu.sync_copy(ones_v, deg_sh.at[idx_v.at[j]], add=True)
        return 0
    lax.fori_loop(0, CPT, body, 0)

    plsc.subcore_barrier()
    pltpu.sync_copy(deg_sh.at[pl.ds(s * 640, 640)], out_hbm.at[c, pl.ds(s * 640, 640)])


@functools.partial(
    pl.kernel,
    out_type=jax.ShapeDtypeStruct((NCORES, NPAD, D), jnp.float32),
    mesh=_mesh,
    scratch_types=[
        pltpu.VMEM_SHARED((NPAD, D), jnp.float32),
        pltpu.VMEM((CPT // 2, CHUNK), jnp.int32),
        pltpu.VMEM((CPT // 2, CHUNK), jnp.int32),
        pltpu.VMEM((2, CHUNK, D), jnp.float32),
        pltpu.VMEM((8, D), jnp.float32),
        pltpu.SemaphoreType.DMA,
        pltpu.SemaphoreType.DMA,
    ],
)
def _acc_kernel(h3_hbm, src_hbm, dst_hbm, out_hbm, acc_sh, sidx_v, didx_v,
                rows_v, zb_v, sem0, sem1):
    c = lax.axis_index("c")
    s = lax.axis_index("s")
    wid = c * NSUB + s

    def zfill(i, _):
        def zrow(k, _):
            zb_v[i, pl.ds(k * 16, 16)] = jnp.zeros((16,), jnp.float32)
            return 0
        lax.fori_loop(0, D // 16, zrow, 0)
        return 0
    lax.fori_loop(0, 8, zfill, 0)

    def zacc(i, _):
        pltpu.sync_copy(zb_v, acc_sh.at[pl.ds(s * (NPAD // NSUB) + i * 8, 8)])
        return 0
    lax.fori_loop(0, NPAD // NSUB // 8, zacc, 0)

    plsc.subcore_barrier()

    sems = (sem0, sem1)
    half = CPT // 2
    for h in range(2):
        pltpu.sync_copy(src_hbm.at[pl.ds(wid * CPT + h * half, half)], sidx_v)
        pltpu.sync_copy(dst_hbm.at[pl.ds(wid * CPT + h * half, half)], didx_v)
        for b in range(2):
            pltpu.async_copy(h3_hbm.at[sidx_v.at[b]], rows_v.at[b], sems[b])

        def body(i, _):
            for b in range(2):
                j = 2 * i + b
                pltpu.make_async_copy(
                    h3_hbm.at[pl.ds(0, CHUNK)], rows_v.at[b], sems[b]).wait()
                pltpu.sync_copy(rows_v.at[b], acc_sh.at[didx_v.at[j]], add=True)
                jn = lax.rem(j + 2, half)
                pltpu.async_copy(h3_hbm.at[sidx_v.at[jn]], rows_v.at[b], sems[b])
            return 0
        lax.fori_loop(0, half // 2, body, 0)
        for b in range(2):
            pltpu.make_async_copy(
                h3_hbm.at[pl.ds(0, CHUNK)], rows_v.at[b], sems[b]).wait()

    plsc.subcore_barrier()
    rpt = NPAD // NSUB
    pltpu.sync_copy(acc_sh.at[pl.ds(s * rpt, rpt)], out_hbm.at[c, pl.ds(s * rpt, rpt)])


def _h2_body(x_ref, w_ref, g_ref, bt_ref, o_ref):
    xb = x_ref[...]
    mean = jnp.mean(xb, axis=1, keepdims=True)
    xc = xb - mean
    var = jnp.mean(xc * xc, axis=1, keepdims=True)
    h = xc * lax.rsqrt(var + 1e-5) * g_ref[...] + bt_ref[...]
    o_ref[...] = jnp.dot(h, w_ref[...], preferred_element_type=jnp.float32)


def _h3_body(h2_ref, degp_ref, o_ref):
    deg = jnp.sum(degp_ref[...], axis=0) + 1.0
    o_ref[...] = h2_ref[...] * lax.rsqrt(deg)[:, None]


def _out_body(x_ref, acc_ref, h3_ref, degp_ref, b_ref, o_ref):
    acc = acc_ref[0] + acc_ref[1] + h3_ref[...]
    deg = jnp.sum(degp_ref[...], axis=0) + 1.0
    g = lax.rsqrt(deg)[:, None] * acc + b_ref[...]
    o_ref[...] = jnp.maximum(x_ref[...] + g, 0.0)


_RB = 2048
_NB = 5

_pad_i = np.arange(EPAD - E, dtype=np.int32)
_PAD_SRC = jnp.asarray(_pad_i % N)
_PAD_DST = jnp.asarray(N + _pad_i % (NPAD - N))


def kernel(x, edge_index, W, b, ln_gamma, ln_beta):
    ei = edge_index.astype(jnp.int32)
    dst2d = jnp.concatenate([ei[1], _PAD_DST]).reshape(NCHUNKS, CHUNK)
    src2d = jnp.concatenate([ei[0], _PAD_SRC]).reshape(NCHUNKS, CHUNK)

    deg_parts = _deg_kernel(dst2d)

    h2 = pl.pallas_call(
        _h2_body,
        grid=(_NB,),
        in_specs=[
            pl.BlockSpec((_RB, D), lambda i: (i, 0)),
            pl.BlockSpec((D, D), lambda i: (0, 0)),
            pl.BlockSpec((1, D), lambda i: (0, 0)),
            pl.BlockSpec((1, D), lambda i: (0, 0)),
        ],
        out_specs=pl.BlockSpec((_RB, D), lambda i: (i, 0)),
        out_shape=jax.ShapeDtypeStruct((N, D), jnp.float32),
    )(x, W, ln_gamma.reshape(1, D), ln_beta.reshape(1, D))

    h3 = pl.pallas_call(
        _h3_body,
        grid=(_NB,),
        in_specs=[
            pl.BlockSpec((_RB, D), lambda i: (i, 0)),
            pl.BlockSpec((NCORES, _RB), lambda i: (0, i)),
        ],
        out_specs=pl.BlockSpec((_RB, D), lambda i: (i, 0)),
        out_shape=jax.ShapeDtypeStruct((N, D), jnp.float32),
    )(h2, deg_parts)

    acc_parts = _acc_kernel(h3, src2d, dst2d)

    out = pl.pallas_call(
        _out_body,
        grid=(_NB,),
        in_specs=[
            pl.BlockSpec((_RB, D), lambda i: (i, 0)),
            pl.BlockSpec((NCORES, _RB, D), lambda i: (0, i, 0)),
            pl.BlockSpec((_RB, D), lambda i: (i, 0)),
            pl.BlockSpec((NCORES, _RB), lambda i: (0, i)),
            pl.BlockSpec((1, D), lambda i: (0, 0)),
        ],
        out_specs=pl.BlockSpec((_RB, D), lambda i: (i, 0)),
        out_shape=jax.ShapeDtypeStruct((N, D), jnp.float32),
    )(x, acc_parts, h3, deg_parts, b.reshape(1, D))

    return out
